# baseline (speedup 1.0000x reference)
import functools
import jax
import jax.numpy as jnp
from jax import lax
from jax.experimental import pallas as pl
from jax.experimental.pallas import tpu as pltpu
from jax.experimental.pallas import tpu_sc as plsc

N_EXP = 16
BLK = 1024
LCH = 128


def _logits_body(x_ref, w_ref, b_ref, lt_ref, cv_ref, esum_ref):
    i = pl.program_id(0)
    nblk = pl.num_programs(0)

    @pl.when(i == 0)
    def _init():
        esum_ref[...] = jnp.zeros_like(esum_ref)

    logits = jnp.dot(x_ref[...], w_ref[...], preferred_element_type=jnp.float32)
    lt = logits.T + b_ref[...]
    lt_ref[...] = lt

    m = jnp.max(lt, axis=0, keepdims=True)
    e = jnp.exp(lt - m)
    s = jnp.sum(e, axis=0, keepdims=True)
    p = e / s
    pc = p.reshape(N_EXP, BLK // LCH, LCH)
    esum_ref[...] += jnp.sum(pc, axis=1)

    @pl.when(i == nblk - 1)
    def _finish():
        sums = jnp.sum(esum_ref[...], axis=1, keepdims=True)
        f = sums / jnp.sum(sums)
        mean = jnp.sum(f) / N_EXP
        var = jnp.sum((f - mean) ** 2) / N_EXP
        cv_ref[...] = jnp.sqrt(var).reshape(1, 1) / mean


def _make_sc_router(n, n_workers, t_per_w):
    mesh = plsc.VectorSubcoreMesh(core_axis_name="c", subcore_axis_name="s")

    @functools.partial(
        pl.kernel, mesh=mesh,
        out_type=[
            jax.ShapeDtypeStruct((2, n), jnp.float32),
            jax.ShapeDtypeStruct((2, n), jnp.int32),
        ],
        scratch_types=[
            pltpu.VMEM((N_EXP, t_per_w), jnp.float32),
            pltpu.VMEM((2, t_per_w), jnp.float32),
            pltpu.VMEM((2, t_per_w), jnp.int32),
        ],
    )
    def sc_router(lt_hbm, wout_hbm, iout_hbm, lt_v, w_v, i_v):
        wid = lax.axis_index("s") * 2 + lax.axis_index("c")
        base = wid * t_per_w
        for ex in range(N_EXP):
            pltpu.sync_copy(lt_hbm.at[ex, pl.ds(base, t_per_w)],
                            lt_v.at[ex, :])

        def group(g, carry):
            t0 = g * 16
            vs = [lt_v[ex, pl.ds(t0, 16)] for ex in range(N_EXP)]
            m1 = vs[0]
            i1 = jnp.zeros((16,), jnp.int32)
            for ex in range(1, N_EXP):
                gt = vs[ex] > m1
                m1 = jnp.where(gt, vs[ex], m1)
                i1 = jnp.where(gt, ex, i1)
            m2 = jnp.full((16,), -jnp.inf, jnp.float32)
            i2 = jnp.zeros((16,), jnp.int32)
            for ex in range(N_EXP):
                ok = (vs[ex] > m2) & (i1 != ex)
                m2 = jnp.where(ok, vs[ex], m2)
                i2 = jnp.where(ok, ex, i2)
            e2 = jnp.exp(m2 - m1)
            w1 = 1.0 / (1.0 + e2)
            w2 = e2 * w1
            w_v[0, pl.ds(t0, 16)] = w1
            w_v[1, pl.ds(t0, 16)] = w2
            i_v[0, pl.ds(t0, 16)] = i1
            i_v[1, pl.ds(t0, 16)] = i2
            return carry

        lax.fori_loop(0, t_per_w // 16, group, 0)

        for r in range(2):
            pltpu.sync_copy(w_v.at[r, :], wout_hbm.at[r, pl.ds(base, t_per_w)])
            pltpu.sync_copy(i_v.at[r, :], iout_hbm.at[r, pl.ds(base, t_per_w)])

    return sc_router


def kernel(x, W, b):
    B, T, d = x.shape
    n = B * T
    x_flat = x.reshape(n, d)
    b2 = b.reshape(N_EXP, 1)
    nblk = n // BLK

    lt_t, cv = pl.pallas_call(
        _logits_body,
        grid=(nblk,),
        in_specs=[
            pl.BlockSpec((BLK, d), lambda i: (i, 0)),
            pl.BlockSpec((d, N_EXP), lambda i: (0, 0)),
            pl.BlockSpec((N_EXP, 1), lambda i: (0, 0)),
        ],
        out_specs=[
            pl.BlockSpec((N_EXP, BLK), lambda i: (0, i)),
            pl.BlockSpec((1, 1), lambda i: (0, 0)),
        ],
        out_shape=[
            jax.ShapeDtypeStruct((N_EXP, n), jnp.float32),
            jax.ShapeDtypeStruct((1, 1), jnp.float32),
        ],
        scratch_shapes=[pltpu.VMEM((N_EXP, LCH), jnp.float32)],
    )(x_flat, W, b2)

    n_workers = 32
    t_per_w = n // n_workers
    wout_t, iout_t = _make_sc_router(n, n_workers, t_per_w)(lt_t)

    wout = wout_t.T.reshape(B, T, 2)
    iout = iout_t.T.reshape(B, T, 2)
    return (wout, iout, cv.reshape(()))


# speedup vs baseline: 1.4795x; 1.4795x over previous
import jax
import jax.numpy as jnp
from jax import lax
from jax.experimental import pallas as pl
from jax.experimental.pallas import tpu as pltpu

N_EXP = 16
BLK = 1024
LCH = 128


def _route(x, w, b, out_ref, esum_ref, col0):
    logits = jnp.dot(x, w, preferred_element_type=jnp.float32)
    lt = logits.T + b

    m = jnp.max(lt, axis=0, keepdims=True)
    e = jnp.exp(lt - m)
    s = jnp.sum(e, axis=0, keepdims=True)
    p = e / s

    pc = p.reshape(N_EXP, BLK // LCH, LCH)
    esum_ref[...] += jnp.sum(pc, axis=1)

    iota = lax.broadcasted_iota(jnp.int32, (N_EXP, BLK), 0)
    eq1 = lt == m
    i1 = jnp.min(jnp.where(eq1, iota, N_EXP), axis=0, keepdims=True)
    l2 = jnp.max(jnp.where(eq1, -jnp.inf, lt), axis=0, keepdims=True)
    i2 = jnp.min(jnp.where(lt == l2, iota, N_EXP), axis=0, keepdims=True)

    e2 = jnp.exp(l2 - m)
    w1 = 1.0 / (1.0 + e2)
    w2 = e2 * w1

    out_ref[0:1, pl.ds(col0, BLK)] = w1
    out_ref[1:2, pl.ds(col0, BLK)] = w2
    out_ref[2:3, pl.ds(col0, BLK)] = i1.astype(jnp.float32)
    out_ref[3:4, pl.ds(col0, BLK)] = i2.astype(jnp.float32)


def _router_body(xa_ref, xb_ref, w_ref, b_ref, out_ref, cv_ref, esum_ref):
    i = pl.program_id(0)
    nblk = pl.num_programs(0)

    @pl.when(i == 0)
    def _init():
        esum_ref[...] = jnp.zeros_like(esum_ref)

    w = w_ref[...]
    b = b_ref[...]
    _route(xa_ref[...], w, b, out_ref, esum_ref, 0)
    _route(xb_ref[...], w, b, out_ref, esum_ref, BLK)

    @pl.when(i == nblk - 1)
    def _finish():
        sums = jnp.sum(esum_ref[...], axis=1, keepdims=True)
        f = sums / jnp.sum(sums)
        mean = jnp.sum(f) / N_EXP
        var = jnp.sum((f - mean) ** 2) / N_EXP
        cv_ref[...] = jnp.sqrt(var).reshape(1, 1) / mean


def kernel(x, W, b):
    B, T, d = x.shape
    n = B * T
    x_flat = x.reshape(n, d)
    b2 = b.reshape(N_EXP, 1)
    nblk = n // (2 * BLK)

    out_t, cv = pl.pallas_call(
        _router_body,
        grid=(nblk,),
        in_specs=[
            pl.BlockSpec((BLK, d), lambda i: (2 * i, 0)),
            pl.BlockSpec((BLK, d), lambda i: (2 * i + 1, 0)),
            pl.BlockSpec((d, N_EXP), lambda i: (0, 0)),
            pl.BlockSpec((N_EXP, 1), lambda i: (0, 0)),
        ],
        out_specs=[
            pl.BlockSpec((4, 2 * BLK), lambda i: (0, i)),
            pl.BlockSpec((1, 1), lambda i: (0, 0)),
        ],
        out_shape=[
            jax.ShapeDtypeStruct((4, n), jnp.float32),
            jax.ShapeDtypeStruct((1, 1), jnp.float32),
        ],
        scratch_shapes=[pltpu.VMEM((N_EXP, LCH), jnp.float32)],
    )(x_flat, x_flat, W, b2)

    o = out_t.T
    wout = o[:, 0:2].reshape(B, T, 2)
    iout = o[:, 2:4].astype(jnp.int32).reshape(B, T, 2)
    return (wout, iout, cv.reshape(()))


# speedup vs baseline: 1.5183x; 1.0262x over previous
import jax
import jax.numpy as jnp
from jax import lax
from jax.experimental import pallas as pl
from jax.experimental.pallas import tpu as pltpu

N_EXP = 16
BLK = 1024
LCH = 128


def _router_body(x_ref, w_ref, b_ref, out_ref, cv_ref, esum_ref):
    i = pl.program_id(0)
    nblk = pl.num_programs(0)

    @pl.when(i == 0)
    def _init():
        esum_ref[...] = jnp.zeros_like(esum_ref)

    logits = jnp.dot(x_ref[...], w_ref[...], preferred_element_type=jnp.float32)
    lt = logits.T + b_ref[...]

    m = jnp.max(lt, axis=0, keepdims=True)
    e = jnp.exp(lt - m)
    s = jnp.sum(e, axis=0, keepdims=True)
    p = e / s

    pc = p.reshape(N_EXP, BLK // LCH, LCH)
    esum_ref[...] += jnp.sum(pc, axis=1)

    iota = lax.broadcasted_iota(jnp.int32, (N_EXP, BLK), 0)
    eq1 = lt == m
    i1 = jnp.min(jnp.where(eq1, iota, N_EXP), axis=0, keepdims=True)
    l2 = jnp.max(jnp.where(eq1, -jnp.inf, lt), axis=0, keepdims=True)
    i2 = jnp.min(jnp.where(lt == l2, iota, N_EXP), axis=0, keepdims=True)

    e2 = jnp.exp(l2 - m)
    w1 = 1.0 / (1.0 + e2)
    w2 = e2 * w1

    out_ref[0:1, :] = w1
    out_ref[1:2, :] = w2
    out_ref[2:3, :] = i1.astype(jnp.float32)
    out_ref[3:4, :] = i2.astype(jnp.float32)

    @pl.when(i == nblk - 1)
    def _finish():
        sums = jnp.sum(esum_ref[...], axis=1, keepdims=True)
        f = sums / jnp.sum(sums)
        mean = jnp.sum(f) / N_EXP
        var = jnp.sum((f - mean) ** 2) / N_EXP
        cv_ref[...] = jnp.sqrt(var).reshape(1, 1) / mean


def kernel(x, W, b):
    B, T, d = x.shape
    n = B * T
    x_flat = x.reshape(n, d)
    b2 = b.reshape(N_EXP, 1)
    nblk = n // BLK

    out_t, cv = pl.pallas_call(
        _router_body,
        grid=(nblk,),
        in_specs=[
            pl.BlockSpec((BLK, d), lambda i: (i, 0)),
            pl.BlockSpec((d, N_EXP), lambda i: (0, 0)),
            pl.BlockSpec((N_EXP, 1), lambda i: (0, 0)),
        ],
        out_specs=[
            pl.BlockSpec((4, BLK), lambda i: (0, i)),
            pl.BlockSpec((1, 1), lambda i: (0, 0)),
        ],
        out_shape=[
            jax.ShapeDtypeStruct((4, n), jnp.float32),
            jax.ShapeDtypeStruct((1, 1), jnp.float32),
        ],
        scratch_shapes=[pltpu.VMEM((N_EXP, LCH), jnp.float32)],
    )(x_flat, W, b2)

    o = out_t.T
    wout = o[:, 0:2].reshape(B, T, 2)
    iout = o[:, 2:4].astype(jnp.int32).reshape(B, T, 2)
    return (wout, iout, cv.reshape(()))


# speedup vs baseline: 1.5250x; 1.0045x over previous
import jax
import jax.numpy as jnp
from jax import lax
from jax.experimental import pallas as pl
from jax.experimental.pallas import tpu as pltpu

N_EXP = 16
BLK = 1024
LCH = 128


def _router_body(x_ref, w_ref, b_ref, out_ref, cv_ref, esum_ref):
    i = pl.program_id(0)
    nblk = pl.num_programs(0)

    @pl.when(i == 0)
    def _init():
        esum_ref[...] = jnp.zeros_like(esum_ref)

    logits = jnp.dot(x_ref[...], w_ref[...], preferred_element_type=jnp.float32)
    lt = logits.T + b_ref[...]

    m = jnp.max(lt, axis=0, keepdims=True)
    e = jnp.exp(lt - m)
    s = jnp.sum(e, axis=0, keepdims=True)
    p = e / s

    pc = p.reshape(N_EXP, BLK // LCH, LCH)
    esum_ref[...] += jnp.sum(pc, axis=1)

    iota = lax.broadcasted_iota(jnp.int32, (N_EXP, BLK), 0)
    eq1 = lt == m
    i1 = jnp.min(jnp.where(eq1, iota, N_EXP), axis=0, keepdims=True)
    not1 = iota != i1
    l2 = jnp.max(jnp.where(not1, lt, -jnp.inf), axis=0, keepdims=True)
    i2 = jnp.min(jnp.where((lt == l2) & not1, iota, N_EXP),
                 axis=0, keepdims=True)

    e2 = jnp.exp(l2 - m)
    w1 = 1.0 / (1.0 + e2)
    w2 = e2 * w1

    out_ref[0:1, :] = w1
    out_ref[1:2, :] = w2
    out_ref[2:3, :] = i1.astype(jnp.float32)
    out_ref[3:4, :] = i2.astype(jnp.float32)

    @pl.when(i == nblk - 1)
    def _finish():
        sums = jnp.sum(esum_ref[...], axis=1, keepdims=True)
        f = sums / jnp.sum(sums)
        mean = jnp.sum(f) / N_EXP
        var = jnp.sum((f - mean) ** 2) / N_EXP
        cv_ref[...] = jnp.sqrt(var).reshape(1, 1) / mean


def kernel(x, W, b):
    B, T, d = x.shape
    n = B * T
    x_flat = x.reshape(n, d)
    b2 = b.reshape(N_EXP, 1)
    nblk = n // BLK

    out_t, cv = pl.pallas_call(
        _router_body,
        grid=(nblk,),
        in_specs=[
            pl.BlockSpec((BLK, d), lambda i: (i, 0)),
            pl.BlockSpec((d, N_EXP), lambda i: (0, 0)),
            pl.BlockSpec((N_EXP, 1), lambda i: (0, 0)),
        ],
        out_specs=[
            pl.BlockSpec((4, BLK), lambda i: (0, i)),
            pl.BlockSpec((1, 1), lambda i: (0, 0)),
        ],
        out_shape=[
            jax.ShapeDtypeStruct((4, n), jnp.float32),
            jax.ShapeDtypeStruct((1, 1), jnp.float32),
        ],
        scratch_shapes=[pltpu.VMEM((N_EXP, LCH), jnp.float32)],
    )(x_flat, W, b2)

    o = out_t.T
    wout = o[:, 0:2].reshape(B, T, 2)
    iout = o[:, 2:4].astype(jnp.int32).reshape(B, T, 2)
    return (wout, iout, cv.reshape(()))
